# baseline (speedup 1.0000x reference)
import jax
import jax.numpy as jnp
from jax import lax
from jax.experimental import pallas as pl

N = 262144
NUM_CLASSES = 80

BLK = 8192
RB = N // BLK


def _roll1(x):
    return jnp.concatenate([x[:, 1:], x[:, :1]], axis=1)


def _body(cls_ref, lab_ref, obj_ref, tobj_ref, obj2_ref, tbb_ref, gbb_ref,
          obj4_ref, cls_out, obj_out, bb_out):
    x = cls_ref[...]
    xT = jnp.transpose(x, (1, 0))
    lab = lab_ref[0]
    gobj = obj_ref[0]
    iota_c = lax.broadcasted_iota(jnp.int32, (NUM_CLASSES, BLK), 0)
    onehot = (iota_c == lab).astype(jnp.float32)
    e = jnp.exp(xT)
    s = jnp.sum(e, axis=0, keepdims=True)
    xt = jnp.sum(xT * onehot, axis=0, keepdims=True)
    logp = xt - jnp.log(s)
    p = jnp.exp(logp)
    f_cls = -(1.0 - p) * (1.0 - p) * logp
    m_cls = (gobj == 1).astype(jnp.float32)

    to = tobj_ref[0]
    g2 = obj2_ref[0]
    tn = _roll1(to)
    so = jnp.exp(to) + jnp.exp(tn)
    olab = jnp.clip(g2, 0, 1)
    xo = jnp.where(olab == 0, to, tn)
    logp_o = xo - jnp.log(so)
    po = jnp.exp(logp_o)
    f_obj = -(1.0 - po) * (1.0 - po) * logp_o
    par2 = lax.broadcasted_iota(jnp.int32, (1, 2 * BLK), 1) & 1
    m_obj = jnp.where((par2 == 0) & (g2 != -1), 1.0, 0.0)

    d = jnp.abs(tbb_ref[0] - gbb_ref[0])
    sl1 = jnp.where(d < 0.1, 0.5 * d * d / 0.1, d - 0.05)
    m_bb = (obj4_ref[0] == 1).astype(jnp.float32)

    @pl.when(pl.program_id(0) == 0)
    def _():
        cls_out[...] = jnp.zeros((1, BLK), jnp.float32)
        obj_out[...] = jnp.zeros((1, 2 * BLK), jnp.float32)
        bb_out[...] = jnp.zeros((1, 4 * BLK), jnp.float32)

    cls_out[...] += f_cls * m_cls
    obj_out[...] += f_obj * m_obj
    bb_out[...] += sl1 * m_bb


def _fused_loss(tcls, gcls3, gobj3, tobj3, gobj23, tbb3, gbb3, gobj43):
    return pl.pallas_call(
        _body,
        grid=(RB,),
        in_specs=[
            pl.BlockSpec((BLK, NUM_CLASSES), lambda i: (i, 0)),
            pl.BlockSpec((1, 1, BLK), lambda i: (i, 0, 0)),
            pl.BlockSpec((1, 1, BLK), lambda i: (i, 0, 0)),
            pl.BlockSpec((1, 1, 2 * BLK), lambda i: (i, 0, 0)),
            pl.BlockSpec((1, 1, 2 * BLK), lambda i: (i, 0, 0)),
            pl.BlockSpec((1, 1, 4 * BLK), lambda i: (i, 0, 0)),
            pl.BlockSpec((1, 1, 4 * BLK), lambda i: (i, 0, 0)),
            pl.BlockSpec((1, 1, 4 * BLK), lambda i: (i, 0, 0)),
        ],
        out_specs=[
            pl.BlockSpec((1, BLK), lambda i: (0, 0)),
            pl.BlockSpec((1, 2 * BLK), lambda i: (0, 0)),
            pl.BlockSpec((1, 4 * BLK), lambda i: (0, 0)),
        ],
        out_shape=[
            jax.ShapeDtypeStruct((1, BLK), jnp.float32),
            jax.ShapeDtypeStruct((1, 2 * BLK), jnp.float32),
            jax.ShapeDtypeStruct((1, 4 * BLK), jnp.float32),
        ],
    )(tcls, gcls3, gobj3, tobj3, gobj23, tbb3, gbb3, gobj43)


def kernel(targets_bb, targets_cls, targets_obj, gt_targets_bb,
           gt_targets_cls, gt_targets_obj, w_objectness, w_class, w_bb, step):
    targets_cls = jnp.reshape(targets_cls, (-1, NUM_CLASSES))
    tbb3 = jnp.reshape(targets_bb, (RB, 1, 4 * BLK))
    tobj3 = jnp.reshape(targets_obj, (RB, 1, 2 * BLK))
    gbb3 = lax.stop_gradient(jnp.reshape(gt_targets_bb, (RB, 1, 4 * BLK)))
    gcls = jnp.reshape(gt_targets_cls, (-1,)).astype(jnp.int32)
    gobj = jnp.reshape(gt_targets_obj, (-1,)).astype(jnp.int32)

    gcls3 = jnp.reshape(gcls, (RB, 1, BLK))
    gobj3 = jnp.reshape(gobj, (RB, 1, BLK))
    gobj23 = jnp.reshape(jnp.repeat(gobj, 2), (RB, 1, 2 * BLK))
    gobj43 = jnp.reshape(jnp.repeat(gobj, 4), (RB, 1, 4 * BLK))

    cls_part, obj_part, bb_part = _fused_loss(
        targets_cls, gcls3, gobj3, tobj3, gobj23, tbb3, gbb3, gobj43)

    num_anchors = jnp.float32(N)
    obj_loss = jnp.sum(obj_part) / num_anchors * 5000.0
    cls_loss = jnp.sum(cls_part) / num_anchors * 10000.0
    bb_loss = jnp.sum(bb_part) / num_anchors * 20000.0

    def _kendall(loss, w):
        return loss * jnp.exp(-w) + w

    return (_kendall(cls_loss, w_class),
            _kendall(obj_loss, w_objectness),
            _kendall(bb_loss, w_bb))
